# baseline (speedup 1.0000x reference)
import random as _pyrandom

import numpy as _np
import jax
import jax.numpy as jnp
from jax.experimental import pallas as pl
from jax.experimental.pallas import tpu as pltpu

_PATCH = 5
_NUM_PATCHES = 20
_L = 2048
_W = 8
_H = 12
_E = 64
_HE = _H * _E
_CHUNK = 512
_NCH = _L // _CHUNK


def _patch_layout():
    rng = _pyrandom.Random(0)
    starts = [rng.randint(0, _L - _PATCH) for _ in range(_NUM_PATCHES)]
    ivs = sorted((s, s + _PATCH) for s in starts)
    clusters = []
    cs, ce = ivs[0]
    for s, e in ivs[1:]:
        if s < ce:
            ce = max(ce, e)
        else:
            clusters.append((cs, ce))
            cs, ce = s, e
    clusters.append((cs, ce))
    C = len(clusters)
    mask = _np.zeros((C * _W, C * _W), _np.float32)
    for ci, (S, _Ec) in enumerate(clusters):
        for i in range(_W):
            r = S + i
            for j in range(_W):
                c = S + j
                if any(s <= r < s + _PATCH and s <= c < s + _PATCH
                       for s in starts):
                    mask[ci * _W + i, ci * _W + j] = 1.0
    return [S for S, _Ec in clusters], mask


_STARTS, _MASK_NP = _patch_layout()
_C = len(_STARTS)
_CW = _C * _W


_ALIGNED = [(S // 8) * 8 for S in _STARTS]
_AW = 16


def _window_copies(q_hbm, k_hbm, qal_ref, kal_ref, sem):
    copies = []
    for ci, base in enumerate(_ALIGNED):
        copies.append(pltpu.make_async_copy(
            q_hbm.at[pl.ds(base, _AW), :], qal_ref.at[pl.ds(ci * _AW, _AW), :],
            sem.at[2 * ci]))
        copies.append(pltpu.make_async_copy(
            k_hbm.at[pl.ds(base, _AW), :], kal_ref.at[pl.ds(ci * _AW, _AW), :],
            sem.at[2 * ci + 1]))
    return copies


def _body(vals_ref, q_hbm, k_hbm, mask_ref, out_ref,
          acc_ref, qal_ref, kal_ref, vw_ref, rows_ref, sem):
    i = pl.program_id(0)

    @pl.when(i == 0)
    def _init():
        acc_ref[...] = jnp.zeros_like(acc_ref)
        for c in _window_copies(q_hbm, k_hbm, qal_ref, kal_ref, sem):
            c.start()

    @pl.when(i < _NCH)
    def _accum():
        acc_ref[...] += jnp.sum(vals_ref[...], axis=0, keepdims=True)

    for ci, S in enumerate(_STARTS):
        ch = S // _CHUNK

        @pl.when(i == ch)
        def _vcopy(ci=ci, S=S, ch=ch):
            vw_ref[ci * _W:(ci + 1) * _W, :] = \
                vals_ref[pl.ds(S - ch * _CHUNK, _W), :]

    @pl.when(i == _NCH - 1)
    def _corrections():
        for c in _window_copies(q_hbm, k_hbm, qal_ref, kal_ref, sem):
            c.wait()
        qparts = []
        kparts = []
        for ci, (S, base) in enumerate(zip(_STARTS, _ALIGNED)):
            off = ci * _AW + (S - base)
            qparts.append(qal_ref[pl.ds(off, _W), :])
            kparts.append(kal_ref[pl.ds(off, _W), :])
        qw = jnp.concatenate(qparts, axis=0)
        kw = jnp.concatenate(kparts, axis=0)
        sv = acc_ref[...]
        mask = mask_ref[...]
        neg = jnp.float32(-1e30)
        n = jnp.sum(mask, axis=1, keepdims=True)
        per_head = []
        for h in range(_H):
            qh = qw[:, h * _E:(h + 1) * _E]
            kh = kw[:, h * _E:(h + 1) * _E]
            vh = vw_ref[:, h * _E:(h + 1) * _E]
            s = jnp.dot(qh, kh.T, preferred_element_type=jnp.float32)
            sm = s * mask + (1.0 - mask) * neg
            m = jnp.maximum(jnp.max(sm, axis=1, keepdims=True), 0.0)
            p = jnp.exp(sm - m)
            sumexp = jnp.sum(p, axis=1, keepdims=True)
            em = jnp.exp(-m)
            z = (jnp.float32(_L) - n) * em + sumexp
            w = p - mask * em
            corr = jnp.dot(w, vh, preferred_element_type=jnp.float32)
            svh = sv[:, h * _E:(h + 1) * _E]
            per_head.append((em * svh + corr) / z)
        rows_ref[...] = jnp.concatenate(per_head, axis=1)

    @pl.when(i >= _NCH - 1)
    def _write_base():
        out_ref[...] = jnp.broadcast_to(acc_ref[...] * (1.0 / _L),
                                        (_CHUNK, _HE))

    for ci, S in enumerate(_STARTS):
        ch = S // _CHUNK

        @pl.when(i == _NCH - 1 + ch)
        def _store(ci=ci, S=S, ch=ch):
            out_ref[pl.ds(S - ch * _CHUNK, _W), :] = \
                rows_ref[ci * _W:(ci + 1) * _W, :]


def kernel(queries, keys, values):
    B, L, H, E = queries.shape
    HE = H * E
    q2 = queries.reshape(L, HE)
    k2 = keys.reshape(L, HE)
    vals2 = values.reshape(L, HE)

    out = pl.pallas_call(
        _body,
        grid=(2 * _NCH - 1,),
        in_specs=[
            pl.BlockSpec((_CHUNK, _HE), lambda i: (jnp.minimum(i, _NCH - 1), 0)),
            pl.BlockSpec(memory_space=pl.ANY),
            pl.BlockSpec(memory_space=pl.ANY),
            pl.BlockSpec((_CW, _CW), lambda i: (0, 0)),
        ],
        out_specs=pl.BlockSpec((_CHUNK, _HE),
                               lambda i: (jnp.maximum(i - (_NCH - 1), 0), 0)),
        out_shape=jax.ShapeDtypeStruct((L, HE), jnp.float32),
        scratch_shapes=[
            pltpu.VMEM((1, _HE), jnp.float32),
            pltpu.VMEM((_C * _AW, _HE), jnp.float32),
            pltpu.VMEM((_C * _AW, _HE), jnp.float32),
            pltpu.VMEM((_CW, _HE), jnp.float32),
            pltpu.VMEM((_CW, _HE), jnp.float32),
            pltpu.SemaphoreType.DMA((2 * _C,)),
        ],
    )(vals2, q2, k2, jnp.asarray(_MASK_NP))

    return (out.reshape(B, L, H, E), None)


# speedup vs baseline: 1.0289x; 1.0289x over previous
import random as _pyrandom

import numpy as _np
import jax
import jax.numpy as jnp
from jax.experimental import pallas as pl
from jax.experimental.pallas import tpu as pltpu

_PATCH = 5
_NUM_PATCHES = 20
_L = 2048
_W = 8
_H = 12
_E = 64
_HE = _H * _E
_CHUNK = 1024
_NCH = _L // _CHUNK


def _patch_layout():
    rng = _pyrandom.Random(0)
    starts = [rng.randint(0, _L - _PATCH) for _ in range(_NUM_PATCHES)]
    ivs = sorted((s, s + _PATCH) for s in starts)
    clusters = []
    cs, ce = ivs[0]
    for s, e in ivs[1:]:
        if s < ce:
            ce = max(ce, e)
        else:
            clusters.append((cs, ce))
            cs, ce = s, e
    clusters.append((cs, ce))
    C = len(clusters)
    mask = _np.zeros((C * _W, C * _W), _np.float32)
    for ci, (S, _Ec) in enumerate(clusters):
        for i in range(_W):
            r = S + i
            for j in range(_W):
                c = S + j
                if any(s <= r < s + _PATCH and s <= c < s + _PATCH
                       for s in starts):
                    mask[ci * _W + i, ci * _W + j] = 1.0
    return [S for S, _Ec in clusters], mask


_STARTS, _MASK_NP = _patch_layout()
_C = len(_STARTS)
_CW = _C * _W


_ALIGNED = [(S // 8) * 8 for S in _STARTS]
_AW = 16


def _window_copies(q_hbm, k_hbm, qal_ref, kal_ref, sem):
    copies = []
    for ci, base in enumerate(_ALIGNED):
        copies.append(pltpu.make_async_copy(
            q_hbm.at[pl.ds(base, _AW), :], qal_ref.at[pl.ds(ci * _AW, _AW), :],
            sem.at[2 * ci]))
        copies.append(pltpu.make_async_copy(
            k_hbm.at[pl.ds(base, _AW), :], kal_ref.at[pl.ds(ci * _AW, _AW), :],
            sem.at[2 * ci + 1]))
    return copies


def _body(vals_ref, q_hbm, k_hbm, mask_ref, out_ref,
          acc_ref, qal_ref, kal_ref, vw_ref, rows_ref, sem):
    i = pl.program_id(0)

    @pl.when(i == 0)
    def _init():
        acc_ref[...] = jnp.zeros_like(acc_ref)
        for c in _window_copies(q_hbm, k_hbm, qal_ref, kal_ref, sem):
            c.start()

    @pl.when(i < _NCH)
    def _accum():
        acc_ref[...] += jnp.sum(vals_ref[...], axis=0, keepdims=True)

    for ci, S in enumerate(_STARTS):
        ch = S // _CHUNK

        @pl.when(i == ch)
        def _vcopy(ci=ci, S=S, ch=ch):
            vw_ref[ci * _W:(ci + 1) * _W, :] = \
                vals_ref[pl.ds(S - ch * _CHUNK, _W), :]

    @pl.when(i == _NCH - 1)
    def _corrections():
        for c in _window_copies(q_hbm, k_hbm, qal_ref, kal_ref, sem):
            c.wait()
        qparts = []
        kparts = []
        for ci, (S, base) in enumerate(zip(_STARTS, _ALIGNED)):
            off = ci * _AW + (S - base)
            qparts.append(qal_ref[pl.ds(off, _W), :])
            kparts.append(kal_ref[pl.ds(off, _W), :])
        qw = jnp.concatenate(qparts, axis=0)
        kw = jnp.concatenate(kparts, axis=0)
        sv = acc_ref[...]
        mask = mask_ref[...]
        neg = jnp.float32(-1e30)
        n = jnp.sum(mask, axis=1, keepdims=True)
        per_head = []
        for h in range(_H):
            qh = qw[:, h * _E:(h + 1) * _E]
            kh = kw[:, h * _E:(h + 1) * _E]
            vh = vw_ref[:, h * _E:(h + 1) * _E]
            s = jnp.dot(qh, kh.T, preferred_element_type=jnp.float32)
            sm = s * mask + (1.0 - mask) * neg
            m = jnp.maximum(jnp.max(sm, axis=1, keepdims=True), 0.0)
            p = jnp.exp(sm - m)
            sumexp = jnp.sum(p, axis=1, keepdims=True)
            em = jnp.exp(-m)
            z = (jnp.float32(_L) - n) * em + sumexp
            w = p - mask * em
            corr = jnp.dot(w, vh, preferred_element_type=jnp.float32)
            svh = sv[:, h * _E:(h + 1) * _E]
            per_head.append((em * svh + corr) / z)
        rows_ref[...] = jnp.concatenate(per_head, axis=1)

    @pl.when(i >= _NCH - 1)
    def _write_base():
        out_ref[...] = jnp.broadcast_to(acc_ref[...] * (1.0 / _L),
                                        (_CHUNK, _HE))

    for ci, S in enumerate(_STARTS):
        ch = S // _CHUNK

        @pl.when(i == _NCH - 1 + ch)
        def _store(ci=ci, S=S, ch=ch):
            out_ref[pl.ds(S - ch * _CHUNK, _W), :] = \
                rows_ref[ci * _W:(ci + 1) * _W, :]


def kernel(queries, keys, values):
    B, L, H, E = queries.shape
    HE = H * E
    q2 = queries.reshape(L, HE)
    k2 = keys.reshape(L, HE)
    vals2 = values.reshape(L, HE)

    out = pl.pallas_call(
        _body,
        grid=(2 * _NCH - 1,),
        in_specs=[
            pl.BlockSpec((_CHUNK, _HE), lambda i: (jnp.minimum(i, _NCH - 1), 0)),
            pl.BlockSpec(memory_space=pl.ANY),
            pl.BlockSpec(memory_space=pl.ANY),
            pl.BlockSpec((_CW, _CW), lambda i: (0, 0)),
        ],
        out_specs=pl.BlockSpec((_CHUNK, _HE),
                               lambda i: (jnp.maximum(i - (_NCH - 1), 0), 0)),
        out_shape=jax.ShapeDtypeStruct((L, HE), jnp.float32),
        scratch_shapes=[
            pltpu.VMEM((1, _HE), jnp.float32),
            pltpu.VMEM((_C * _AW, _HE), jnp.float32),
            pltpu.VMEM((_C * _AW, _HE), jnp.float32),
            pltpu.VMEM((_CW, _HE), jnp.float32),
            pltpu.VMEM((_CW, _HE), jnp.float32),
            pltpu.SemaphoreType.DMA((2 * _C,)),
        ],
    )(vals2, q2, k2, jnp.asarray(_MASK_NP))

    return (out.reshape(B, L, H, E), None)
